# baseline (speedup 1.0000x reference)
import functools
import math

import jax
import jax.numpy as jnp
import numpy as np
from jax import lax
from jax.experimental import pallas as pl
from jax.experimental.pallas import tpu as pltpu
from jax.experimental.pallas import tpu_sc as plsc

VOCAB = 1000000
D = 64
SEQ = 200
SCALE = math.sqrt(D)

NC = 2
NS = 16
NW = NC * NS

CB = 400


def _pe_template(rows: int) -> np.ndarray:
    position = np.arange(SEQ, dtype=np.float32)[:, None]
    num_even = D // 2 + D % 2
    div_term = np.exp(
        np.arange(0, num_even, dtype=np.float32) * (-math.log(10000.0) / D)
    )
    pe = np.zeros((SEQ, D), dtype=np.float32)
    pe[:, 0::2] = np.sin(position * div_term[:num_even])
    pe[:, 1::2] = np.cos(position * div_term[: D // 2])
    return np.tile(pe, (rows // SEQ, 1)).astype(np.float32)


def _sc_embed(x_flat, table, pe_tile, n_rows):
    chunks_per_w = n_rows // (NW * CB)
    b_per_w = n_rows // NW
    mesh = plsc.VectorSubcoreMesh(
        core_axis_name="c", subcore_axis_name="s", num_cores=NC, num_subcores=NS
    )

    @functools.partial(
        pl.kernel,
        out_type=jax.ShapeDtypeStruct((n_rows, D), jnp.float32),
        mesh=mesh,
        compiler_params=pltpu.CompilerParams(use_tc_tiling_on_sc=False),
        scratch_types=[
            pltpu.VMEM((CB,), jnp.int32),
            pltpu.VMEM((CB, D), jnp.float32),
            pltpu.VMEM((CB, D), jnp.float32),
            pltpu.SemaphoreType.DMA,
        ],
    )
    def k(x_hbm, table_hbm, pe_hbm, out_hbm, idx_v, rows_v, pe_v, sem):
        wid = lax.axis_index("s") * NC + lax.axis_index("c")
        base = wid * b_per_w
        pltpu.sync_copy(pe_hbm, pe_v)

        @pl.loop(0, chunks_per_w)
        def _chunk(c):
            off = base + c * CB
            pltpu.sync_copy(x_hbm.at[pl.ds(off, CB)], idx_v)
            pltpu.async_copy(table_hbm.at[idx_v], rows_v, sem).wait()

            @pl.loop(0, CB)
            def _row(r):
                for j in range(D // 16):
                    s = pl.ds(j * 16, 16)
                    rows_v[r, s] = rows_v[r, s] * SCALE + pe_v[r, s]

            pltpu.sync_copy(rows_v, out_hbm.at[pl.ds(off, CB)])

    return k(x_flat, table, pe_tile)


def kernel(x, table):
    b, s = x.shape
    n_rows = b * s
    x_flat = x.reshape(n_rows).astype(jnp.int32)
    pe_tile = jnp.asarray(_pe_template(CB))
    out = _sc_embed(x_flat, table, pe_tile, n_rows)
    return out.reshape(b, s, D)
